# baseline (speedup 1.0000x reference)
import functools

import jax
import jax.numpy as jnp
from jax import lax
from jax.experimental import pallas as pl
from jax.experimental.pallas import tpu as pltpu
from jax.experimental.pallas import tpu_sc as plsc

EPS = 0.01
H, W = 352, 1216
HW = H * W
RADIUS_PX = 4.0
WS = float(min(H, W))
RADIUS = RADIUS_PX / float(max(H, W)) * 2.0
TAPS = ((-1, -1), (-1, 0), (-1, 1), (0, -1), (0, 0), (0, 1), (1, -1), (1, 0), (1, 1))

BAND_ROWS = 16
BAND_PX = BAND_ROWS * W
NBANDS = H // BAND_ROWS
NB_PER_CORE = NBANDS // 2
NSUB = 16
CH = 65536 // NSUB
STRIPE = BAND_PX // NSUB
R = 64
FLUSH_HI = 512
PEND_CAP = FLUSH_HI + CH + 2 * R
WQ_SCALE = 8192.0
FLAT_MASK = 0x7FFFF


def _phase_a_body(sx_ref, sy_ref, sz_ref, i0_ref, j0_ref, pk_ref):
    sx = sx_ref[0, 0]
    sy = sy_ref[0, 0]
    sz = sz_ref[0, 0]
    i0 = i0_ref[0, 0]
    j0 = j0_ref[0, 0]
    r2 = RADIUS * RADIUS
    words = []
    for di, dj in TAPS:
        ii = i0 + di
        jj = j0 + dj
        xc = (W - 1.0 - 2.0 * jj.astype(jnp.float32)) / WS
        yc = (H - 1.0 - 2.0 * ii.astype(jnp.float32)) / WS
        d2 = (sx - xc) ** 2 + (sy - yc) ** 2
        inside = ((d2 < r2) & (ii >= 0) & (ii < H) & (jj >= 0) & (jj < W)
                  & (sz > 0.0))
        dist = d2 / r2
        alpha = 1.0 - jnp.sqrt(jnp.clip(dist, 0.001, 1.0))
        wq = jnp.round(alpha * WQ_SCALE).astype(jnp.int32)
        flat = (jnp.clip(ii, 0, H - 1) * W + jnp.clip(jj, 0, W - 1))
        word = jnp.where(inside, flat | (wq << 19), HW)
        words.append(word)
    pk_ref[0, 0] = jnp.stack(words, axis=0)


def _project(pts3D, K):
    nK = jnp.zeros_like(K)
    nK = nK.at[:, 0, :].set(K[:, 0, :] / WS)
    nK = nK.at[:, 1, :].set(K[:, 1, :] / WS)
    nK = nK.at[:, 2, 2].set(1.0)
    xy_proj = jnp.einsum('bij,bjn->bin', nK, pts3D)
    mask = jnp.abs(xy_proj[:, 2:3, :]) < EPS
    zs = jnp.where(mask, EPS, xy_proj[:, 2:3, :])
    sampler = jnp.concatenate([
        2.0 * xy_proj[:, 0:1, :] / -zs + W / WS,
        2.0 * xy_proj[:, 1:2, :] / -zs + H / WS,
        xy_proj[:, 2:3, :]], axis=1)
    sampler = jnp.where(jnp.broadcast_to(mask, sampler.shape), -10.0, sampler)
    sx = sampler[:, 0:1, :]
    sy = sampler[:, 1:2, :]
    sz = sampler[:, 2:3, :]
    j0 = jnp.round((W - 1.0 - sx * WS) / 2.0).astype(jnp.int32)
    i0 = jnp.round((H - 1.0 - sy * WS) / 2.0).astype(jnp.int32)
    return sx, sy, sz, i0, j0


def _phase_a(pts3D, K):
    B = pts3D.shape[0]
    N = pts3D.shape[2]
    sx, sy, sz, i0, j0 = _project(pts3D, K)
    grid = (B, NSUB)
    pk = pl.pallas_call(
        _phase_a_body,
        grid=grid,
        in_specs=[
            pl.BlockSpec((1, 1, CH), lambda b, s: (b, 0, s)),
            pl.BlockSpec((1, 1, CH), lambda b, s: (b, 0, s)),
            pl.BlockSpec((1, 1, CH), lambda b, s: (b, 0, s)),
            pl.BlockSpec((1, 1, CH), lambda b, s: (b, 0, s)),
            pl.BlockSpec((1, 1, CH), lambda b, s: (b, 0, s)),
        ],
        out_specs=pl.BlockSpec((1, 1, 9, CH), lambda b, s: (b, s, 0, 0)),
        out_shape=jax.ShapeDtypeStruct((B, NSUB, 9, CH), jnp.int32),
    )(sx, sy, sz, i0, j0)
    return pk


def _sc_splat(pk, feat2, B, N):
    mesh = plsc.VectorSubcoreMesh(core_axis_name="c", subcore_axis_name="s")

    @functools.partial(
        pl.kernel,
        mesh=mesh,
        out_type=jax.ShapeDtypeStruct((B * HW, 64), jnp.float32),
        compiler_params=pltpu.CompilerParams(
            use_tc_tiling_on_sc=False, needs_layout_passes=False),
        scratch_types=[
            pltpu.VMEM((9, CH), jnp.int32),
            pltpu.VMEM((PEND_CAP,), jnp.int32),
            pltpu.VMEM((R,), jnp.int32),
            pltpu.VMEM((R,), jnp.int32),
            pltpu.VMEM((R,), jnp.float32),
            pltpu.VMEM((R, 64), jnp.float32),
            pltpu.VMEM((R, 64), jnp.float32),
            pltpu.VMEM_SHARED((BAND_PX + 8, 64), jnp.float32),
            pltpu.SemaphoreType.DMA,
        ],
    )
    def k(pk_hbm, feat_hbm, out_hbm, words, pend, idx_sub, pid_sub, w_sub,
          rows, zbuf, acc, sem):
        cid = lax.axis_index("c")
        sid = lax.axis_index("s")
        col0 = sid * CH
        zf16 = jnp.zeros((16,), jnp.float32)
        lane = jnp.arange(16, dtype=jnp.int32)
        dnums = lax.GatherDimensionNumbers(
            offset_dims=(), collapsed_slice_dims=(0,), start_index_map=(0,))

        def bcast_lane(vec, l):
            idx = jnp.full((16, 1), l, jnp.int32)
            return lax.gather(vec, idx, dimension_numbers=dnums,
                              slice_sizes=(1,),
                              mode=lax.GatherScatterMode.PROMISE_IN_BOUNDS)

        def zb_body(r, _):
            for c4 in range(4):
                zbuf[r, pl.ds(c4 * 16, 16)] = zf16
            return 0
        lax.fori_loop(0, R, zb_body, 0)

        def band_body(it, _):
            b = it // NB_PER_CORE
            band = (it % NB_PER_CORE) * 2 + cid
            lo = band * BAND_PX
            hi = lo + BAND_PX
            pid_base = b * N + col0

            def z_body(kz, _):
                pltpu.sync_copy(zbuf, acc.at[pl.ds(sid * STRIPE + kz * R, R)])
                return 0
            lax.fori_loop(0, STRIPE // R, z_body, 0)
            pltpu.sync_copy(pk_hbm.at[b, sid], words)
            plsc.subcore_barrier()

            def flush_if(cnt, thresh):
                nb = jnp.where(cnt >= thresh, (cnt + (R - 1)) // R, 0)

                def j_body(j, _):
                    off = j * R
                    for q in range(R // 16):
                        gpos = off + q * 16 + lane
                        mv = gpos < cnt
                        p16 = pend[pl.ds(off + q * 16, 16)]
                        wd = plsc.load_gather(
                            words, [p16 >> 12, p16 & (CH - 1)], mask=mv)
                        fl = wd & FLAT_MASK
                        wv = (((wd >> 19) & 0x1FFF).astype(jnp.float32)
                              / WQ_SCALE)
                        idx_sub[pl.ds(q * 16, 16)] = jnp.where(mv, fl - lo,
                                                               BAND_PX)
                        pid_sub[pl.ds(q * 16, 16)] = jnp.where(
                            mv, pid_base + (p16 & (CH - 1)), 0)
                        w_sub[pl.ds(q * 16, 16)] = jnp.where(mv, wv, 0.0)
                    pltpu.async_copy(feat_hbm.at[pid_sub], rows, sem).wait()

                    def q_body(q, _):
                        w16 = w_sub[pl.ds(q * 16, 16)]
                        for l in range(16):
                            wb = bcast_lane(w16, l)
                            ri = q * 16 + l
                            for c4 in range(4):
                                rows[ri, pl.ds(c4 * 16, 16)] = (
                                    rows[ri, pl.ds(c4 * 16, 16)] * wb)
                        return 0
                    lax.fori_loop(0, R // 16, q_body, 0)
                    pltpu.sync_copy(rows, acc.at[idx_sub], add=True)
                    return 0
                lax.fori_loop(0, nb, j_body, 0)
                return jnp.where(nb > 0, 0, cnt)

            def row_body(t, cnt):
                def g_body(g, cnt):
                    g16 = g * 16
                    wd = words[t, pl.ds(g16, 16)]
                    fl = wd & FLAT_MASK
                    m = (fl >= lo) & (fl < hi)
                    pop = jnp.sum(m.astype(jnp.int32))

                    def do_compact(c):
                        incl = plsc.cumsum(m.astype(jnp.int32))
                        plsc.store_scatter(pend, [c + incl - 1],
                                           t * CH + g16 + lane, mask=m)
                        return c + pop
                    return lax.cond(pop > 0, do_compact, lambda c: c, cnt)
                cnt = lax.fori_loop(0, CH // 16, g_body, cnt)
                return flush_if(cnt, FLUSH_HI)

            cnt = lax.fori_loop(0, 9, row_body, jnp.int32(0))
            flush_if(cnt, 1)
            plsc.subcore_barrier()

            def wb_body(kz, _):
                row0 = sid * STRIPE + kz * R
                pltpu.sync_copy(acc.at[pl.ds(row0, R)],
                                out_hbm.at[pl.ds(b * HW + lo + row0, R)])
                return 0
            lax.fori_loop(0, STRIPE // R, wb_body, 0)
            plsc.subcore_barrier()
            return 0

        lax.fori_loop(0, B * NB_PER_CORE, band_body, 0)

    return k(pk, feat2)


def kernel(src_feat, pts3D, K):
    B, C, N = src_feat.shape
    pk = _phase_a(pts3D, K)
    feat2 = jnp.transpose(src_feat, (0, 2, 1)).reshape(B * N, C)
    out = _sc_splat(pk, feat2, B, N)
    return jnp.transpose(out.reshape(B, H, W, C), (0, 3, 1, 2))


# speedup vs baseline: 1.3137x; 1.3137x over previous
import functools

import jax
import jax.numpy as jnp
from jax import lax
from jax.experimental import pallas as pl
from jax.experimental.pallas import tpu as pltpu
from jax.experimental.pallas import tpu_sc as plsc

EPS = 0.01
H, W = 352, 1216
HW = H * W
RADIUS_PX = 4.0
WS = float(min(H, W))
RADIUS = RADIUS_PX / float(max(H, W)) * 2.0
TAPS = ((-1, -1), (-1, 0), (-1, 1), (0, -1), (0, 0), (0, 1), (1, -1), (1, 0), (1, 1))

BAND_ROWS = 16
BAND_PX = BAND_ROWS * W
NBANDS = H // BAND_ROWS
NB_PER_CORE = NBANDS // 2
NSUB = 16
CH = 65536 // NSUB
STRIPE = BAND_PX // NSUB
R = 64
FLUSH_HI = 512
PEND_CAP = FLUSH_HI + CH + 2 * R
WQ_SCALE = 8192.0
FLAT_MASK = 0x7FFFF


def _phase_a_body(sx_ref, sy_ref, sz_ref, i0_ref, j0_ref, pk_ref):
    sx = sx_ref[0, 0]
    sy = sy_ref[0, 0]
    sz = sz_ref[0, 0]
    i0 = i0_ref[0, 0]
    j0 = j0_ref[0, 0]
    r2 = RADIUS * RADIUS
    words = []
    for di, dj in TAPS:
        ii = i0 + di
        jj = j0 + dj
        xc = (W - 1.0 - 2.0 * jj.astype(jnp.float32)) / WS
        yc = (H - 1.0 - 2.0 * ii.astype(jnp.float32)) / WS
        d2 = (sx - xc) ** 2 + (sy - yc) ** 2
        inside = ((d2 < r2) & (ii >= 0) & (ii < H) & (jj >= 0) & (jj < W)
                  & (sz > 0.0))
        dist = d2 / r2
        alpha = 1.0 - jnp.sqrt(jnp.clip(dist, 0.001, 1.0))
        wq = jnp.round(alpha * WQ_SCALE).astype(jnp.int32)
        flat = (jnp.clip(ii, 0, H - 1) * W + jnp.clip(jj, 0, W - 1))
        word = jnp.where(inside, flat | (wq << 19), HW)
        words.append(word)
    pk_ref[0, 0] = jnp.stack(words, axis=0)


def _project(pts3D, K):
    nK = jnp.zeros_like(K)
    nK = nK.at[:, 0, :].set(K[:, 0, :] / WS)
    nK = nK.at[:, 1, :].set(K[:, 1, :] / WS)
    nK = nK.at[:, 2, 2].set(1.0)
    xy_proj = jnp.einsum('bij,bjn->bin', nK, pts3D)
    mask = jnp.abs(xy_proj[:, 2:3, :]) < EPS
    zs = jnp.where(mask, EPS, xy_proj[:, 2:3, :])
    sampler = jnp.concatenate([
        2.0 * xy_proj[:, 0:1, :] / -zs + W / WS,
        2.0 * xy_proj[:, 1:2, :] / -zs + H / WS,
        xy_proj[:, 2:3, :]], axis=1)
    sampler = jnp.where(jnp.broadcast_to(mask, sampler.shape), -10.0, sampler)
    sx = sampler[:, 0:1, :]
    sy = sampler[:, 1:2, :]
    sz = sampler[:, 2:3, :]
    j0 = jnp.round((W - 1.0 - sx * WS) / 2.0).astype(jnp.int32)
    i0 = jnp.round((H - 1.0 - sy * WS) / 2.0).astype(jnp.int32)
    return sx, sy, sz, i0, j0


def _phase_a(pts3D, K):
    B = pts3D.shape[0]
    N = pts3D.shape[2]
    sx, sy, sz, i0, j0 = _project(pts3D, K)
    grid = (B, NSUB)
    pk = pl.pallas_call(
        _phase_a_body,
        grid=grid,
        in_specs=[
            pl.BlockSpec((1, 1, CH), lambda b, s: (b, 0, s)),
            pl.BlockSpec((1, 1, CH), lambda b, s: (b, 0, s)),
            pl.BlockSpec((1, 1, CH), lambda b, s: (b, 0, s)),
            pl.BlockSpec((1, 1, CH), lambda b, s: (b, 0, s)),
            pl.BlockSpec((1, 1, CH), lambda b, s: (b, 0, s)),
        ],
        out_specs=pl.BlockSpec((1, 1, 9, CH), lambda b, s: (b, s, 0, 0)),
        out_shape=jax.ShapeDtypeStruct((B, NSUB, 9, CH), jnp.int32),
    )(sx, sy, sz, i0, j0)
    return pk


def _sc_splat(pk, feat2, B, N):
    mesh = plsc.VectorSubcoreMesh(core_axis_name="c", subcore_axis_name="s")

    @functools.partial(
        pl.kernel,
        mesh=mesh,
        out_type=jax.ShapeDtypeStruct((B * HW, 64), jnp.float32),
        compiler_params=pltpu.CompilerParams(
            use_tc_tiling_on_sc=False, needs_layout_passes=False),
        scratch_types=[
            pltpu.VMEM((9, CH), jnp.int32),
            pltpu.VMEM((PEND_CAP,), jnp.int32),
            pltpu.VMEM((R,), jnp.int32),
            pltpu.VMEM((R,), jnp.int32),
            pltpu.VMEM((R,), jnp.float32),
            pltpu.VMEM((R, 64), jnp.float32),
            pltpu.VMEM((R, 64), jnp.float32),
            pltpu.VMEM_SHARED((BAND_PX + 8, 64), jnp.float32),
            pltpu.SemaphoreType.DMA,
        ],
    )
    def k(pk_hbm, feat_hbm, out_hbm, words, pend, idx_sub, pid_sub, w_sub,
          rows, zbuf, acc, sem):
        cid = lax.axis_index("c")
        sid = lax.axis_index("s")
        col0 = sid * CH
        zf16 = jnp.zeros((16,), jnp.float32)
        lane = jnp.arange(16, dtype=jnp.int32)
        dnums = lax.GatherDimensionNumbers(
            offset_dims=(), collapsed_slice_dims=(0,), start_index_map=(0,))

        def bcast_lane(vec, l):
            idx = jnp.full((16, 1), l, jnp.int32)
            return lax.gather(vec, idx, dimension_numbers=dnums,
                              slice_sizes=(1,),
                              mode=lax.GatherScatterMode.PROMISE_IN_BOUNDS)

        def zb_body(r, _):
            for c4 in range(4):
                zbuf[r, pl.ds(c4 * 16, 16)] = zf16
            return 0
        lax.fori_loop(0, R, zb_body, 0)

        def band_body(it, _):
            b = it // NB_PER_CORE
            band = (it % NB_PER_CORE) * 2 + cid
            lo = band * BAND_PX
            hi = lo + BAND_PX
            pid_base = b * N + col0

            def z_body(kz, _):
                pltpu.sync_copy(zbuf, acc.at[pl.ds(sid * STRIPE + kz * R, R)])
                return 0
            lax.fori_loop(0, STRIPE // R, z_body, 0)
            pltpu.sync_copy(pk_hbm.at[b, sid], words)
            plsc.subcore_barrier()

            def flush_if(cnt, thresh):
                nb = jnp.where(cnt >= thresh, (cnt + (R - 1)) // R, 0)

                def j_body(j, _):
                    off = j * R
                    for q in range(R // 16):
                        gpos = off + q * 16 + lane
                        mv = gpos < cnt
                        p16 = pend[pl.ds(off + q * 16, 16)]
                        wd = plsc.load_gather(
                            words, [p16 >> 12, p16 & (CH - 1)], mask=mv)
                        fl = wd & FLAT_MASK
                        wv = (((wd >> 19) & 0x1FFF).astype(jnp.float32)
                              / WQ_SCALE)
                        idx_sub[pl.ds(q * 16, 16)] = jnp.where(mv, fl - lo,
                                                               BAND_PX)
                        pid_sub[pl.ds(q * 16, 16)] = jnp.where(
                            mv, pid_base + (p16 & (CH - 1)), 0)
                        w_sub[pl.ds(q * 16, 16)] = jnp.where(mv, wv, 0.0)
                    pltpu.async_copy(feat_hbm.at[pid_sub], rows, sem).wait()

                    def q_body(q, _):
                        w16 = w_sub[pl.ds(q * 16, 16)]
                        for l in range(16):
                            wb = bcast_lane(w16, l)
                            ri = q * 16 + l
                            for c4 in range(4):
                                rows[ri, pl.ds(c4 * 16, 16)] = (
                                    rows[ri, pl.ds(c4 * 16, 16)] * wb)
                        return 0
                    lax.fori_loop(0, R // 16, q_body, 0)
                    pltpu.sync_copy(rows, acc.at[idx_sub], add=True)
                    return 0
                lax.fori_loop(0, nb, j_body, 0)
                return jnp.where(nb > 0, 0, cnt)

            def row_body(t, cnt):
                def g_body(g, cnt):
                    g16 = g * 16
                    wd = words[t, pl.ds(g16, 16)]
                    fl = wd & FLAT_MASK
                    m = (fl >= lo) & (fl < hi)
                    mi = m.astype(jnp.int32)
                    incl = plsc.cumsum(mi)
                    plsc.store_scatter(pend, [cnt + incl - 1],
                                       t * CH + g16 + lane, mask=m)
                    return cnt + jnp.sum(mi)
                cnt = lax.fori_loop(0, CH // 16, g_body, cnt)
                return flush_if(cnt, FLUSH_HI)

            cnt = lax.fori_loop(0, 9, row_body, jnp.int32(0))
            flush_if(cnt, 1)
            plsc.subcore_barrier()

            def wb_body(kz, _):
                row0 = sid * STRIPE + kz * R
                pltpu.sync_copy(acc.at[pl.ds(row0, R)],
                                out_hbm.at[pl.ds(b * HW + lo + row0, R)])
                return 0
            lax.fori_loop(0, STRIPE // R, wb_body, 0)
            plsc.subcore_barrier()
            return 0

        lax.fori_loop(0, B * NB_PER_CORE, band_body, 0)

    return k(pk, feat2)


def kernel(src_feat, pts3D, K):
    B, C, N = src_feat.shape
    pk = _phase_a(pts3D, K)
    feat2 = jnp.transpose(src_feat, (0, 2, 1)).reshape(B * N, C)
    out = _sc_splat(pk, feat2, B, N)
    return jnp.transpose(out.reshape(B, H, W, C), (0, 3, 1, 2))


# speedup vs baseline: 1.5551x; 1.1837x over previous
import functools

import jax
import jax.numpy as jnp
from jax import lax
from jax.experimental import pallas as pl
from jax.experimental.pallas import tpu as pltpu
from jax.experimental.pallas import tpu_sc as plsc

EPS = 0.01
H, W = 352, 1216
HW = H * W
RADIUS_PX = 4.0
WS = float(min(H, W))
RADIUS = RADIUS_PX / float(max(H, W)) * 2.0
TAPS = ((-1, -1), (-1, 0), (-1, 1), (0, -1), (0, 0), (0, 1), (1, -1), (1, 0), (1, 1))

BAND_ROWS = 16
BAND_PX = BAND_ROWS * W
NBANDS = H // BAND_ROWS
NB_PER_CORE = NBANDS // 2
NSUB = 16
CH = 65536 // NSUB
STRIPE = BAND_PX // NSUB
R = 64
FLUSH_HI = 512
PEND_CAP = FLUSH_HI + CH + 2 * R
WQ_SCALE = 8192.0
FLAT_MASK = 0x7FFFF


def _phase_a_body(sx_ref, sy_ref, sz_ref, i0_ref, j0_ref, pk_ref):
    sx = sx_ref[0, 0]
    sy = sy_ref[0, 0]
    sz = sz_ref[0, 0]
    i0 = i0_ref[0, 0]
    j0 = j0_ref[0, 0]
    r2 = RADIUS * RADIUS
    words = []
    for di, dj in TAPS:
        ii = i0 + di
        jj = j0 + dj
        xc = (W - 1.0 - 2.0 * jj.astype(jnp.float32)) / WS
        yc = (H - 1.0 - 2.0 * ii.astype(jnp.float32)) / WS
        d2 = (sx - xc) ** 2 + (sy - yc) ** 2
        inside = ((d2 < r2) & (ii >= 0) & (ii < H) & (jj >= 0) & (jj < W)
                  & (sz > 0.0))
        dist = d2 / r2
        alpha = 1.0 - jnp.sqrt(jnp.clip(dist, 0.001, 1.0))
        wq = jnp.round(alpha * WQ_SCALE).astype(jnp.int32)
        flat = (jnp.clip(ii, 0, H - 1) * W + jnp.clip(jj, 0, W - 1))
        word = jnp.where(inside, flat | (wq << 19), HW)
        words.append(word)
    pk_ref[0, 0] = jnp.stack(words, axis=0)


def _project(pts3D, K):
    nK = jnp.zeros_like(K)
    nK = nK.at[:, 0, :].set(K[:, 0, :] / WS)
    nK = nK.at[:, 1, :].set(K[:, 1, :] / WS)
    nK = nK.at[:, 2, 2].set(1.0)
    xy_proj = jnp.einsum('bij,bjn->bin', nK, pts3D)
    mask = jnp.abs(xy_proj[:, 2:3, :]) < EPS
    zs = jnp.where(mask, EPS, xy_proj[:, 2:3, :])
    sampler = jnp.concatenate([
        2.0 * xy_proj[:, 0:1, :] / -zs + W / WS,
        2.0 * xy_proj[:, 1:2, :] / -zs + H / WS,
        xy_proj[:, 2:3, :]], axis=1)
    sampler = jnp.where(jnp.broadcast_to(mask, sampler.shape), -10.0, sampler)
    sx = sampler[:, 0:1, :]
    sy = sampler[:, 1:2, :]
    sz = sampler[:, 2:3, :]
    j0 = jnp.round((W - 1.0 - sx * WS) / 2.0).astype(jnp.int32)
    i0 = jnp.round((H - 1.0 - sy * WS) / 2.0).astype(jnp.int32)
    return sx, sy, sz, i0, j0


def _phase_a(pts3D, K):
    B = pts3D.shape[0]
    N = pts3D.shape[2]
    sx, sy, sz, i0, j0 = _project(pts3D, K)
    grid = (B, NSUB)
    pk = pl.pallas_call(
        _phase_a_body,
        grid=grid,
        in_specs=[
            pl.BlockSpec((1, 1, CH), lambda b, s: (b, 0, s)),
            pl.BlockSpec((1, 1, CH), lambda b, s: (b, 0, s)),
            pl.BlockSpec((1, 1, CH), lambda b, s: (b, 0, s)),
            pl.BlockSpec((1, 1, CH), lambda b, s: (b, 0, s)),
            pl.BlockSpec((1, 1, CH), lambda b, s: (b, 0, s)),
        ],
        out_specs=pl.BlockSpec((1, 1, 9, CH), lambda b, s: (b, s, 0, 0)),
        out_shape=jax.ShapeDtypeStruct((B, NSUB, 9, CH), jnp.int32),
    )(sx, sy, sz, i0, j0)
    return pk


def _sc_splat(pk, feat2, B, N):
    mesh = plsc.VectorSubcoreMesh(core_axis_name="c", subcore_axis_name="s")

    @functools.partial(
        pl.kernel,
        mesh=mesh,
        out_type=jax.ShapeDtypeStruct((B * HW, 64), jnp.float32),
        compiler_params=pltpu.CompilerParams(
            use_tc_tiling_on_sc=False, needs_layout_passes=False),
        scratch_types=[
            pltpu.VMEM((9, CH), jnp.int32),
            pltpu.VMEM((PEND_CAP,), jnp.int32),
            pltpu.VMEM((R,), jnp.int32),
            pltpu.VMEM((R,), jnp.int32),
            pltpu.VMEM((R,), jnp.float32),
            pltpu.VMEM((R, 64), jnp.float32),
            pltpu.VMEM((R, 64), jnp.float32),
            pltpu.VMEM_SHARED((BAND_PX + 8, 64), jnp.float32),
            pltpu.SemaphoreType.DMA,
        ],
    )
    def k(pk_hbm, feat_hbm, out_hbm, words, pend, idx_sub, pid_sub, w_sub,
          rows, zbuf, acc, sem):
        cid = lax.axis_index("c")
        sid = lax.axis_index("s")
        col0 = sid * CH
        zf16 = jnp.zeros((16,), jnp.float32)
        lane = jnp.arange(16, dtype=jnp.int32)
        dnums = lax.GatherDimensionNumbers(
            offset_dims=(), collapsed_slice_dims=(0,), start_index_map=(0,))

        def bcast_lane(vec, l):
            idx = jnp.full((16, 1), l, jnp.int32)
            return lax.gather(vec, idx, dimension_numbers=dnums,
                              slice_sizes=(1,),
                              mode=lax.GatherScatterMode.PROMISE_IN_BOUNDS)

        def zb_body(r, _):
            for c4 in range(4):
                zbuf[r, pl.ds(c4 * 16, 16)] = zf16
            return 0
        lax.fori_loop(0, R, zb_body, 0)

        def band_body(it, _):
            b = it // NB_PER_CORE
            band = (it % NB_PER_CORE) * 2 + cid
            lo = band * BAND_PX
            hi = lo + BAND_PX
            pid_base = b * N + col0

            def z_body(kz, _):
                pltpu.sync_copy(zbuf, acc.at[pl.ds(sid * STRIPE + kz * R, R)])
                return 0
            lax.fori_loop(0, STRIPE // R, z_body, 0)
            pltpu.sync_copy(pk_hbm.at[b, sid], words)
            plsc.subcore_barrier()

            def flush_if(cnt, thresh):
                nb = jnp.where(cnt >= thresh, (cnt + (R - 1)) // R, 0)

                def j_body(j, _):
                    off = j * R
                    for q in range(R // 16):
                        gpos = off + q * 16 + lane
                        mv = gpos < cnt
                        p16 = pend[pl.ds(off + q * 16, 16)]
                        wd = plsc.load_gather(
                            words, [p16 >> 12, p16 & (CH - 1)], mask=mv)
                        fl = wd & FLAT_MASK
                        wv = (((wd >> 19) & 0x1FFF).astype(jnp.float32)
                              / WQ_SCALE)
                        idx_sub[pl.ds(q * 16, 16)] = jnp.where(mv, fl - lo,
                                                               BAND_PX)
                        pid_sub[pl.ds(q * 16, 16)] = jnp.where(
                            mv, pid_base + (p16 & (CH - 1)), 0)
                        w_sub[pl.ds(q * 16, 16)] = jnp.where(mv, wv, 0.0)
                    pltpu.async_copy(feat_hbm.at[pid_sub], rows, sem).wait()

                    def q_body(q, _):
                        w16 = w_sub[pl.ds(q * 16, 16)]
                        for l in range(16):
                            wb = bcast_lane(w16, l)
                            ri = q * 16 + l
                            for c4 in range(4):
                                rows[ri, pl.ds(c4 * 16, 16)] = (
                                    rows[ri, pl.ds(c4 * 16, 16)] * wb)
                        return 0
                    lax.fori_loop(0, R // 16, q_body, 0)
                    pltpu.sync_copy(rows, acc.at[idx_sub], add=True)
                    return 0
                lax.fori_loop(0, nb, j_body, 0)
                return jnp.where(nb > 0, 0, cnt)

            def row_body(t, cnt):
                def g_body(g, cnt):
                    g64 = g * 64
                    incls = []
                    masks = []
                    for u in range(4):
                        wd = words[t, pl.ds(g64 + u * 16, 16)]
                        fl = wd & FLAT_MASK
                        m = (fl >= lo) & (fl < hi)
                        incls.append(plsc.cumsum(m.astype(jnp.int32)))
                        masks.append(m)
                    for u in range(4):
                        plsc.store_scatter(pend, [cnt + incls[u] - 1],
                                           t * CH + g64 + u * 16 + lane,
                                           mask=masks[u])
                        cnt = cnt + incls[u][15]
                    return cnt
                cnt = lax.fori_loop(0, CH // 64, g_body, cnt)
                return flush_if(cnt, FLUSH_HI)

            cnt = lax.fori_loop(0, 9, row_body, jnp.int32(0))
            flush_if(cnt, 1)
            plsc.subcore_barrier()

            def wb_body(kz, _):
                row0 = sid * STRIPE + kz * R
                pltpu.sync_copy(acc.at[pl.ds(row0, R)],
                                out_hbm.at[pl.ds(b * HW + lo + row0, R)])
                return 0
            lax.fori_loop(0, STRIPE // R, wb_body, 0)
            plsc.subcore_barrier()
            return 0

        lax.fori_loop(0, B * NB_PER_CORE, band_body, 0)

    return k(pk, feat2)


def kernel(src_feat, pts3D, K):
    B, C, N = src_feat.shape
    pk = _phase_a(pts3D, K)
    feat2 = jnp.transpose(src_feat, (0, 2, 1)).reshape(B * N, C)
    out = _sc_splat(pk, feat2, B, N)
    return jnp.transpose(out.reshape(B, H, W, C), (0, 3, 1, 2))
